# baseline (speedup 1.0000x reference)
import functools

import jax
import jax.numpy as jnp
from jax import lax
from jax.experimental import pallas as pl
from jax.experimental.pallas import tpu as pltpu
from jax.experimental.pallas import tpu_sc as plsc

LANES = 16
BLK = 128


@functools.lru_cache(maxsize=None)
def _build_gather(vocab, embed, batch, seq, nc, ns):
    nw = nc * ns
    mesh = plsc.VectorSubcoreMesh(core_axis_name="c", subcore_axis_name="s")
    b_per_w = batch // nw
    ntb = b_per_w // BLK
    et = embed // 8
    n_units = seq * ntb
    n_iter = n_units // 2
    assert b_per_w % BLK == 0 and embed % 8 == 0 and n_iter * 2 == n_units

    @functools.partial(
        pl.kernel,
        out_type=jax.ShapeDtypeStruct(
            (seq, et, batch // BLK, 8 * BLK), jnp.float32),
        mesh=mesh,
        scratch_types=[
            pltpu.VMEM((BLK,), jnp.int32),
            pltpu.VMEM((BLK,), jnp.int32),
            pltpu.VMEM((BLK, embed), jnp.float32),
            pltpu.VMEM((BLK, embed), jnp.float32),
            pltpu.VMEM((embed * BLK,), jnp.float32),
            pltpu.VMEM((embed * BLK,), jnp.float32),
            pltpu.SemaphoreType.DMA,
            pltpu.SemaphoreType.DMA,
            pltpu.SemaphoreType.DMA,
            pltpu.SemaphoreType.DMA,
            pltpu.SemaphoreType.DMA,
            pltpu.SemaphoreType.DMA,
            pltpu.SemaphoreType.DMA,
            pltpu.SemaphoreType.DMA,
        ],
        compiler_params=pltpu.CompilerParams(
            use_tc_tiling_on_sc=False, needs_layout_passes=False,
            disable_bounds_checks=True),
    )
    def gather(table_hbm, idxt_hbm, out_hbm, idxb0, idxb1,
               rows0, rows1, tile0, tile1, gs0, gs1, ws0, ws1, is0, is1,
               ts0, ts1):
        wid = lax.axis_index("s") * nc + lax.axis_index("c")
        ii = lax.iota(jnp.int32, LANES)

        def unit_coords(n):
            s = n // ntb
            tb = lax.rem(n, ntb)
            return s, wid * ntb + tb

        def fetch_idx(n, idxb, isem):
            s, tbg = unit_coords(n)
            pltpu.async_copy(idxt_hbm.at[s, pl.ds(BLK * tbg, BLK)], idxb, isem)

        def wait_idx(idxb, isem):
            pltpu.make_async_copy(idxt_hbm.at[0, pl.ds(0, BLK)], idxb, isem).wait()

        def fire_gather(idxb, rows, gsem):
            pltpu.async_copy(table_hbm.at[idxb], rows, gsem)

        def drain_gather(rows, gsem):
            pltpu.make_async_copy(table_hbm.at[pl.ds(0, BLK)], rows, gsem).wait()

        pmod = [lax.rem(ii + d, LANES) for d in range(LANES)]
        vstore = [pmod[d] * BLK + ii for d in range(LANES)]

        def transpose(rows, tile, tsem):
            del tsem
            for g in range(BLK // LANES):
                for h in range(embed // LANES):
                    c_s = LANES * h * BLK + LANES * g
                    for d in range(LANES):
                        v = plsc.load_gather(
                            rows, [LANES * g + ii, LANES * h + pmod[d]])
                        plsc.store_scatter(tile, [vstore[d] + c_s], v)

        def drain_transpose(tile, tsem):
            del tile, tsem

        def fire_writes(n, tile, wsem):
            s, tbg = unit_coords(n)
            for e8 in range(et):
                pltpu.async_copy(tile.at[pl.ds(8 * BLK * e8, 8 * BLK)],
                                 out_hbm.at[s, e8, tbg], wsem)

        def drain_writes(tile, wsem):
            for e8 in range(et):
                pltpu.make_async_copy(
                    out_hbm.at[0, 0, 0],
                    tile.at[pl.ds(8 * BLK * e8, 8 * BLK)], wsem).wait()

        fetch_idx(0, idxb0, is0)
        fetch_idx(1, idxb1, is1)
        wait_idx(idxb0, is0)
        fire_gather(idxb0, rows0, gs0)
        wait_idx(idxb1, is1)
        fire_gather(idxb1, rows1, gs1)

        def stage_a(t, n, idxb, rows, tile, gsem, wsem, isem, tsem):
            drain_gather(rows, gsem)

            @pl.when(t < n_iter - 1)
            def _():
                fetch_idx(n + 2, idxb, isem)

            @pl.when(t >= 1)
            def _():
                drain_writes(tile, wsem)

            transpose(rows, tile, tsem)

        def stage_b(t, n, idxb, rows, tile, gsem, wsem, isem, tsem):
            drain_transpose(tile, tsem)
            fire_writes(n, tile, wsem)

            @pl.when(t < n_iter - 1)
            def _():
                wait_idx(idxb, isem)
                fire_gather(idxb, rows, gsem)

        def body(t, carry):
            n0 = 2 * t
            stage_a(t, n0, idxb0, rows0, tile0, gs0, ws0, is0, ts0)
            stage_a(t, n0 + 1, idxb1, rows1, tile1, gs1, ws1, is1, ts1)
            stage_b(t, n0, idxb0, rows0, tile0, gs0, ws0, is0, ts0)
            stage_b(t, n0 + 1, idxb1, rows1, tile1, gs1, ws1, is1, ts1)
            return carry

        lax.fori_loop(0, n_iter, body, 0)
        drain_writes(tile0, ws0)
        drain_writes(tile1, ws1)

    return gather


def kernel(indices, table):
    b, s = indices.shape
    vocab, embed = table.shape
    info = plsc.get_sparse_core_info()
    nc, ns = info.num_cores, info.num_subcores
    idx_t = indices.astype(jnp.int32).T
    out4 = _build_gather(vocab, embed, b, s, nc, ns)(table, idx_t)
    out5 = out4.reshape(s, embed // 8, b // BLK, 8, BLK)
    return jnp.transpose(out5, (2, 4, 0, 1, 3)).reshape(b, s, embed)


# speedup vs baseline: 1.0398x; 1.0398x over previous
import functools

import jax
import jax.numpy as jnp
from jax import lax
from jax.experimental import pallas as pl
from jax.experimental.pallas import tpu as pltpu
from jax.experimental.pallas import tpu_sc as plsc

LANES = 16
BLK = 128


@functools.lru_cache(maxsize=None)
def _build_gather(vocab, embed, batch, seq, nc, ns):
    nw = nc * ns
    mesh = plsc.VectorSubcoreMesh(core_axis_name="c", subcore_axis_name="s")
    b_per_w = batch // nw
    ntb = b_per_w // BLK
    et = embed // 8
    n_units = seq * ntb
    n_iter = n_units // 2
    assert b_per_w % BLK == 0 and embed % 8 == 0 and n_iter * 2 == n_units

    @functools.partial(
        pl.kernel,
        out_type=jax.ShapeDtypeStruct(
            (seq, et, batch // BLK, 8 * BLK), jnp.float32),
        mesh=mesh,
        scratch_types=[
            pltpu.VMEM((BLK,), jnp.int32),
            pltpu.VMEM((BLK,), jnp.int32),
            pltpu.VMEM((BLK, embed), jnp.float32),
            pltpu.VMEM((BLK, embed), jnp.float32),
            pltpu.VMEM((embed * BLK,), jnp.float32),
            pltpu.VMEM((embed * BLK,), jnp.float32),
            pltpu.SemaphoreType.DMA,
            pltpu.SemaphoreType.DMA,
            pltpu.SemaphoreType.DMA,
            pltpu.SemaphoreType.DMA,
            pltpu.SemaphoreType.DMA,
            pltpu.SemaphoreType.DMA,
        ],
        compiler_params=pltpu.CompilerParams(
            use_tc_tiling_on_sc=False, needs_layout_passes=False,
            disable_bounds_checks=True),
    )
    def gather(table_hbm, idxt_hbm, out_hbm, idxb0, idxb1,
               rows0, rows1, tile0, tile1, gs0, gs1, ws0, ws1, is0, is1):
        wid = lax.axis_index("s") * nc + lax.axis_index("c")
        ii = lax.iota(jnp.int32, LANES)

        def unit_coords(n):
            s = n // ntb
            tb = lax.rem(n, ntb)
            return s, wid * ntb + tb

        def fetch_idx(n, idxb, isem):
            s, tbg = unit_coords(n)
            pltpu.async_copy(idxt_hbm.at[s, pl.ds(BLK * tbg, BLK)], idxb, isem)

        def wait_idx(idxb, isem):
            pltpu.make_async_copy(idxt_hbm.at[0, pl.ds(0, BLK)], idxb, isem).wait()

        def fire_gather(idxb, rows, gsem):
            pltpu.async_copy(table_hbm.at[idxb], rows, gsem)

        def drain_gather(rows, gsem):
            pltpu.make_async_copy(table_hbm.at[pl.ds(0, BLK)], rows, gsem).wait()

        pmod = [lax.rem(ii + d, LANES) for d in range(LANES)]
        vstore = [pmod[d] * BLK + ii for d in range(LANES)]

        def transpose(rows, tile):
            for g in range(BLK // LANES):
                for h in range(embed // LANES):
                    c_s = LANES * h * BLK + LANES * g
                    for d in range(LANES):
                        v = plsc.load_gather(
                            rows, [LANES * g + ii, LANES * h + pmod[d]])
                        plsc.store_scatter(tile, [vstore[d] + c_s], v)

        def fire_writes(n, tile, wsem):
            s, tbg = unit_coords(n)
            for e8 in range(et):
                pltpu.async_copy(tile.at[pl.ds(8 * BLK * e8, 8 * BLK)],
                                 out_hbm.at[s, e8, tbg], wsem)

        def drain_writes(tile, wsem):
            for e8 in range(et):
                pltpu.make_async_copy(
                    out_hbm.at[0, 0, 0],
                    tile.at[pl.ds(8 * BLK * e8, 8 * BLK)], wsem).wait()

        fetch_idx(0, idxb0, is0)
        fetch_idx(1, idxb1, is1)
        wait_idx(idxb0, is0)
        fire_gather(idxb0, rows0, gs0)
        wait_idx(idxb1, is1)
        fire_gather(idxb1, rows1, gs1)

        def half(t, n, idxb, rows, tile, gsem, wsem, isem):
            drain_gather(rows, gsem)

            @pl.when(t < n_iter - 1)
            def _():
                fetch_idx(n + 2, idxb, isem)

            @pl.when(t >= 1)
            def _():
                drain_writes(tile, wsem)

            transpose(rows, tile)
            fire_writes(n, tile, wsem)

            @pl.when(t < n_iter - 1)
            def _():
                wait_idx(idxb, isem)
                fire_gather(idxb, rows, gsem)

        def body(t, carry):
            n0 = 2 * t
            half(t, n0, idxb0, rows0, tile0, gs0, ws0, is0)
            half(t, n0 + 1, idxb1, rows1, tile1, gs1, ws1, is1)
            return carry

        lax.fori_loop(0, n_iter, body, 0)
        drain_writes(tile0, ws0)
        drain_writes(tile1, ws1)

    return gather


def kernel(indices, table):
    b, s = indices.shape
    vocab, embed = table.shape
    info = plsc.get_sparse_core_info()
    nc, ns = info.num_cores, info.num_subcores
    idx_t = indices.astype(jnp.int32).T
    out4 = _build_gather(vocab, embed, b, s, nc, ns)(table, idx_t)
    out5 = out4.reshape(s, embed // 8, b // BLK, 8, BLK)
    return jnp.transpose(out5, (2, 4, 0, 1, 3)).reshape(b, s, embed)


# speedup vs baseline: 1.0414x; 1.0015x over previous
import functools

import jax
import jax.numpy as jnp
from jax import lax
from jax.experimental import pallas as pl
from jax.experimental.pallas import tpu as pltpu
from jax.experimental.pallas import tpu_sc as plsc

LANES = 16
BLK = 128


@functools.lru_cache(maxsize=None)
def _build_gather(vocab, embed, batch, seq, nc, ns):
    nw = nc * ns
    mesh = plsc.VectorSubcoreMesh(core_axis_name="c", subcore_axis_name="s")
    b_per_w = batch // nw
    ntb = b_per_w // BLK
    et = embed // 8
    n_units = seq * ntb
    n_iter = n_units // 2
    assert b_per_w % BLK == 0 and embed % 8 == 0 and n_iter * 2 == n_units

    @functools.partial(
        pl.kernel,
        out_type=jax.ShapeDtypeStruct(
            (seq, et, batch // BLK, 8 * BLK), jnp.float32),
        mesh=mesh,
        scratch_types=[
            pltpu.VMEM((BLK,), jnp.int32),
            pltpu.VMEM((BLK,), jnp.int32),
            pltpu.VMEM((BLK, embed), jnp.float32),
            pltpu.VMEM((BLK, embed), jnp.float32),
            pltpu.VMEM((et, 8 * BLK), jnp.float32),
            pltpu.VMEM((et, 8 * BLK), jnp.float32),
            pltpu.SemaphoreType.DMA,
            pltpu.SemaphoreType.DMA,
            pltpu.SemaphoreType.DMA,
            pltpu.SemaphoreType.DMA,
            pltpu.SemaphoreType.DMA,
            pltpu.SemaphoreType.DMA,
        ],
        compiler_params=pltpu.CompilerParams(
            use_tc_tiling_on_sc=False, needs_layout_passes=False,
            disable_bounds_checks=True),
    )
    def gather(table_hbm, idxt_hbm, out_hbm, idxb0, idxb1,
               rows0, rows1, tile0, tile1, gs0, gs1, ws0, ws1, is0, is1):
        wid = lax.axis_index("s") * nc + lax.axis_index("c")
        ii = lax.iota(jnp.int32, LANES)

        def unit_coords(n):
            s = n // ntb
            tb = lax.rem(n, ntb)
            return s, wid * ntb + tb

        def fetch_idx(n, idxb, isem):
            s, tbg = unit_coords(n)
            pltpu.async_copy(idxt_hbm.at[s, pl.ds(BLK * tbg, BLK)], idxb, isem)

        def wait_idx(idxb, isem):
            pltpu.make_async_copy(idxt_hbm.at[0, pl.ds(0, BLK)], idxb, isem).wait()

        def fire_gather(idxb, rows, gsem):
            pltpu.async_copy(table_hbm.at[idxb], rows, gsem)

        def drain_gather(rows, gsem):
            pltpu.make_async_copy(table_hbm.at[pl.ds(0, BLK)], rows, gsem).wait()

        pmod = [lax.rem(ii + d, LANES) for d in range(LANES)]
        vrow = [lax.shift_right_logical(pmod[d], 3) for d in range(LANES)]
        vcol = [lax.bitwise_and(pmod[d], 7) * BLK + ii for d in range(LANES)]

        def transpose(rows, tile):
            for g in range(BLK // LANES):
                for h in range(embed // LANES):
                    for d in range(LANES):
                        v = plsc.load_gather(
                            rows, [LANES * g + ii, LANES * h + pmod[d]])
                        plsc.store_scatter(
                            tile, [vrow[d] + 2 * h, vcol[d] + LANES * g], v)

        def fire_writes(n, tile, wsem):
            s, tbg = unit_coords(n)
            pltpu.async_copy(tile, out_hbm.at[s, :, tbg], wsem)

        def drain_writes(tile, wsem):
            pltpu.make_async_copy(out_hbm.at[0, :, 0], tile, wsem).wait()

        fetch_idx(0, idxb0, is0)
        fetch_idx(1, idxb1, is1)
        wait_idx(idxb0, is0)
        fire_gather(idxb0, rows0, gs0)
        wait_idx(idxb1, is1)
        fire_gather(idxb1, rows1, gs1)

        def half(t, n, idxb, rows, tile, gsem, wsem, isem):
            drain_gather(rows, gsem)

            @pl.when(t < n_iter - 1)
            def _():
                fetch_idx(n + 2, idxb, isem)

            @pl.when(t >= 1)
            def _():
                drain_writes(tile, wsem)

            transpose(rows, tile)
            fire_writes(n, tile, wsem)

            @pl.when(t < n_iter - 1)
            def _():
                wait_idx(idxb, isem)
                fire_gather(idxb, rows, gsem)

        def body(t, carry):
            n0 = 2 * t
            half(t, n0, idxb0, rows0, tile0, gs0, ws0, is0)
            half(t, n0 + 1, idxb1, rows1, tile1, gs1, ws1, is1)
            return carry

        lax.fori_loop(0, n_iter, body, 0)
        drain_writes(tile0, ws0)
        drain_writes(tile1, ws1)

    return gather


def kernel(indices, table):
    b, s = indices.shape
    vocab, embed = table.shape
    info = plsc.get_sparse_core_info()
    nc, ns = info.num_cores, info.num_subcores
    idx_t = indices.astype(jnp.int32).T
    out4 = _build_gather(vocab, embed, b, s, nc, ns)(table, idx_t)
    out5 = out4.reshape(s, embed // 8, b // BLK, 8, BLK)
    return jnp.transpose(out5, (2, 4, 0, 1, 3)).reshape(b, s, embed)


# speedup vs baseline: 1.2628x; 1.2126x over previous
import functools

import jax
import jax.numpy as jnp
from jax import lax
from jax.experimental import pallas as pl
from jax.experimental.pallas import tpu as pltpu
from jax.experimental.pallas import tpu_sc as plsc

LANES = 16
BLK = 128
UPP = 2


@functools.lru_cache(maxsize=None)
def _build_gather(vocab, embed, batch, seq, nc, ns):
    nw = nc * ns
    mesh = plsc.VectorSubcoreMesh(core_axis_name="c", subcore_axis_name="s")
    b_per_w = batch // nw
    ntb = b_per_w // BLK
    et = embed // 8
    n_pairs = seq * ntb // UPP
    n_iter = n_pairs // 2
    assert b_per_w % BLK == 0 and embed % 8 == 0 and ntb % UPP == 0
    assert n_iter * 2 == n_pairs

    @functools.partial(
        pl.kernel,
        out_type=jax.ShapeDtypeStruct(
            (seq, et, (batch // BLK) * 8 * BLK), jnp.float32),
        mesh=mesh,
        scratch_types=[
            pltpu.VMEM((UPP * BLK,), jnp.int32),
            pltpu.VMEM((UPP * BLK,), jnp.int32),
            pltpu.VMEM((UPP * BLK, embed), jnp.float32),
            pltpu.VMEM((UPP * BLK, embed), jnp.float32),
            pltpu.VMEM((et, UPP * 8 * BLK), jnp.float32),
            pltpu.VMEM((et, UPP * 8 * BLK), jnp.float32),
            pltpu.SemaphoreType.DMA,
            pltpu.SemaphoreType.DMA,
            pltpu.SemaphoreType.DMA,
            pltpu.SemaphoreType.DMA,
            pltpu.SemaphoreType.DMA,
            pltpu.SemaphoreType.DMA,
        ],
        compiler_params=pltpu.CompilerParams(
            use_tc_tiling_on_sc=False, needs_layout_passes=False,
            disable_bounds_checks=True),
    )
    def gather(table_hbm, idxt_hbm, out_hbm, idxb0, idxb1,
               rows0, rows1, tile0, tile1, gs0, gs1, ws0, ws1, is0, is1):
        wid = lax.axis_index("s") * nc + lax.axis_index("c")
        ii = lax.iota(jnp.int32, LANES)

        def pair_coords(p):
            s = p // (ntb // UPP)
            tb0 = UPP * lax.rem(p, ntb // UPP)
            return s, wid * ntb + tb0

        def fetch_idx(p, idxb, isem):
            s, tbg0 = pair_coords(p)
            pltpu.async_copy(
                idxt_hbm.at[s, pl.ds(BLK * tbg0, UPP * BLK)], idxb, isem)

        def wait_idx(idxb, isem):
            pltpu.make_async_copy(
                idxt_hbm.at[0, pl.ds(0, UPP * BLK)], idxb, isem).wait()

        def fire_gathers(idxb, rows, gsem):
            for u in range(UPP):
                pltpu.async_copy(
                    table_hbm.at[idxb.at[pl.ds(u * BLK, BLK)]],
                    rows.at[pl.ds(u * BLK, BLK)], gsem)

        def drain_gathers(rows, gsem):
            pltpu.make_async_copy(
                table_hbm.at[pl.ds(0, UPP * BLK)], rows, gsem).wait()

        pmod = [lax.rem(ii + d, LANES) for d in range(LANES)]
        vrow = [lax.shift_right_logical(pmod[d], 3) for d in range(LANES)]
        vcol = [lax.bitwise_and(pmod[d], 7) * BLK + ii for d in range(LANES)]

        def transpose(rows, tile):
            def unit_body(u, carry):
                for g in range(BLK // LANES):
                    roff = u * BLK + LANES * g
                    coff = u * 8 * BLK + LANES * g
                    for h in range(embed // LANES):
                        for d in range(LANES):
                            v = plsc.load_gather(
                                rows, [roff + ii, LANES * h + pmod[d]])
                            plsc.store_scatter(
                                tile, [vrow[d] + 2 * h, vcol[d] + coff], v)
                return carry

            lax.fori_loop(0, UPP, unit_body, 0)

        def fire_write(p, tile, wsem):
            s, tbg0 = pair_coords(p)
            pltpu.async_copy(
                tile, out_hbm.at[s, :, pl.ds(8 * BLK * tbg0, UPP * 8 * BLK)],
                wsem)

        def drain_write(tile, wsem):
            pltpu.make_async_copy(
                out_hbm.at[0, :, pl.ds(0, UPP * 8 * BLK)], tile, wsem).wait()

        fetch_idx(0, idxb0, is0)
        fetch_idx(1, idxb1, is1)
        wait_idx(idxb0, is0)
        fire_gathers(idxb0, rows0, gs0)
        wait_idx(idxb1, is1)
        fire_gathers(idxb1, rows1, gs1)

        def half(t, p, idxb, rows, tile, gsem, wsem, isem):
            drain_gathers(rows, gsem)

            @pl.when(t < n_iter - 1)
            def _():
                fetch_idx(p + 2, idxb, isem)

            @pl.when(t >= 1)
            def _():
                drain_write(tile, wsem)

            transpose(rows, tile)
            fire_write(p, tile, wsem)

            @pl.when(t < n_iter - 1)
            def _():
                wait_idx(idxb, isem)
                fire_gathers(idxb, rows, gsem)

        def body(t, carry):
            p0 = 2 * t
            half(t, p0, idxb0, rows0, tile0, gs0, ws0, is0)
            half(t, p0 + 1, idxb1, rows1, tile1, gs1, ws1, is1)
            return carry

        lax.fori_loop(0, n_iter, body, 0)
        drain_write(tile0, ws0)
        drain_write(tile1, ws1)

    return gather


def kernel(indices, table):
    b, s = indices.shape
    vocab, embed = table.shape
    info = plsc.get_sparse_core_info()
    nc, ns = info.num_cores, info.num_subcores
    idx_t = indices.astype(jnp.int32).T
    out3 = _build_gather(vocab, embed, b, s, nc, ns)(table, idx_t)
    out5 = out3.reshape(s, embed // 8, b // BLK, 8, BLK)
    return jnp.transpose(out5, (2, 4, 0, 1, 3)).reshape(b, s, embed)


# speedup vs baseline: 1.3342x; 1.0566x over previous
import functools

import jax
import jax.numpy as jnp
from jax import lax
from jax.experimental import pallas as pl
from jax.experimental.pallas import tpu as pltpu
from jax.experimental.pallas import tpu_sc as plsc

LANES = 16
BLK = 128
UPP = 2


@functools.lru_cache(maxsize=None)
def _build_gather(vocab, embed, batch, seq, nc, ns):
    nw = nc * ns
    mesh = plsc.VectorSubcoreMesh(core_axis_name="c", subcore_axis_name="s")
    b_per_w = batch // nw
    ntb = b_per_w // BLK
    et = embed // 8
    n_pairs = seq * ntb // UPP
    n_iter = n_pairs // 2
    assert b_per_w % BLK == 0 and embed % 8 == 0 and ntb % UPP == 0
    assert n_iter * 2 == n_pairs

    @functools.partial(
        pl.kernel,
        out_type=jax.ShapeDtypeStruct(
            (seq, et, (batch // BLK) * 8 * BLK), jnp.float32),
        mesh=mesh,
        scratch_types=[
            pltpu.VMEM((UPP * BLK,), jnp.int32),
            pltpu.VMEM((UPP * BLK,), jnp.int32),
            pltpu.VMEM((UPP * BLK, embed), jnp.float32),
            pltpu.VMEM((UPP * BLK, embed), jnp.float32),
            pltpu.VMEM((et, UPP * 8 * BLK), jnp.float32),
            pltpu.VMEM((et, UPP * 8 * BLK), jnp.float32),
            pltpu.SemaphoreType.DMA,
            pltpu.SemaphoreType.DMA,
            pltpu.SemaphoreType.DMA,
            pltpu.SemaphoreType.DMA,
            pltpu.SemaphoreType.DMA,
            pltpu.SemaphoreType.DMA,
        ],
        compiler_params=pltpu.CompilerParams(
            use_tc_tiling_on_sc=False, needs_layout_passes=False,
            disable_bounds_checks=True),
    )
    def gather(table_hbm, idxt_hbm, out_hbm, idxb0, idxb1,
               rows0, rows1, tile0, tile1, gs0, gs1, ws0, ws1, is0, is1):
        wid = lax.axis_index("s") * nc + lax.axis_index("c")
        ii = lax.iota(jnp.int32, LANES)

        def pair_coords(p):
            s = p // (ntb // UPP)
            tb0 = UPP * lax.rem(p, ntb // UPP)
            return s, wid * ntb + tb0

        def fetch_idx(p, idxb, isem):
            s, tbg0 = pair_coords(p)
            pltpu.async_copy(
                idxt_hbm.at[s, pl.ds(BLK * tbg0, UPP * BLK)], idxb, isem)

        def wait_idx(idxb, isem):
            pltpu.make_async_copy(
                idxt_hbm.at[0, pl.ds(0, UPP * BLK)], idxb, isem).wait()

        def fire_gathers(idxb, rows, gsem):
            for u in range(UPP):
                pltpu.async_copy(
                    table_hbm.at[idxb.at[pl.ds(u * BLK, BLK)]],
                    rows.at[pl.ds(u * BLK, BLK)], gsem)

        def drain_gathers(rows, gsem):
            pltpu.make_async_copy(
                table_hbm.at[pl.ds(0, UPP * BLK)], rows, gsem).wait()

        pmod = [lax.rem(ii + d, LANES) for d in range(LANES)]
        vrow = [lax.shift_right_logical(pmod[d], 3) for d in range(LANES)]
        vcol = [lax.bitwise_and(pmod[d], 7) * BLK + ii for d in range(LANES)]

        def transpose(rows, tile):
            def blk_body(ug, carry):
                u = ug // (BLK // LANES)
                g = lax.rem(ug, BLK // LANES)
                roff = u * BLK + LANES * g
                coff = u * 8 * BLK + LANES * g
                rvec = roff + ii
                for h in range(embed // LANES):
                    for d in range(LANES):
                        v = plsc.load_gather(
                            rows, [rvec, LANES * h + pmod[d]])
                        plsc.store_scatter(
                            tile, [vrow[d] + 2 * h, vcol[d] + coff], v)
                return carry

            lax.fori_loop(0, UPP * (BLK // LANES), blk_body, 0)

        def fire_write(p, tile, wsem):
            s, tbg0 = pair_coords(p)
            pltpu.async_copy(
                tile, out_hbm.at[s, :, pl.ds(8 * BLK * tbg0, UPP * 8 * BLK)],
                wsem)

        def drain_write(tile, wsem):
            pltpu.make_async_copy(
                out_hbm.at[0, :, pl.ds(0, UPP * 8 * BLK)], tile, wsem).wait()

        fetch_idx(0, idxb0, is0)
        fetch_idx(1, idxb1, is1)
        wait_idx(idxb0, is0)
        fire_gathers(idxb0, rows0, gs0)
        wait_idx(idxb1, is1)
        fire_gathers(idxb1, rows1, gs1)

        def half(t, p, idxb, rows, tile, gsem, wsem, isem):
            drain_gathers(rows, gsem)

            @pl.when(t < n_iter - 1)
            def _():
                fetch_idx(p + 2, idxb, isem)

            @pl.when(t >= 1)
            def _():
                drain_write(tile, wsem)

            transpose(rows, tile)
            fire_write(p, tile, wsem)

            @pl.when(t < n_iter - 1)
            def _():
                wait_idx(idxb, isem)
                fire_gathers(idxb, rows, gsem)

        def body(t, carry):
            p0 = 2 * t
            half(t, p0, idxb0, rows0, tile0, gs0, ws0, is0)
            half(t, p0 + 1, idxb1, rows1, tile1, gs1, ws1, is1)
            return carry

        lax.fori_loop(0, n_iter, body, 0)
        drain_write(tile0, ws0)
        drain_write(tile1, ws1)

    return gather


def kernel(indices, table):
    b, s = indices.shape
    vocab, embed = table.shape
    info = plsc.get_sparse_core_info()
    nc, ns = info.num_cores, info.num_subcores
    idx_t = indices.astype(jnp.int32).T
    out3 = _build_gather(vocab, embed, b, s, nc, ns)(table, idx_t)
    out5 = out3.reshape(s, embed // 8, b // BLK, 8, BLK)
    return jnp.transpose(out5, (2, 4, 0, 1, 3)).reshape(b, s, embed)


# speedup vs baseline: 1.3389x; 1.0035x over previous
import functools

import jax
import jax.numpy as jnp
from jax import lax
from jax.experimental import pallas as pl
from jax.experimental.pallas import tpu as pltpu
from jax.experimental.pallas import tpu_sc as plsc

LANES = 16
BLK = 128
UPP = 4


@functools.lru_cache(maxsize=None)
def _build_gather(vocab, embed, batch, seq, nc, ns):
    nw = nc * ns
    mesh = plsc.VectorSubcoreMesh(core_axis_name="c", subcore_axis_name="s")
    b_per_w = batch // nw
    ntb = b_per_w // BLK
    et = embed // 8
    n_pairs = seq * ntb // UPP
    n_iter = n_pairs // 2
    assert b_per_w % BLK == 0 and embed % 8 == 0 and ntb % UPP == 0
    assert n_iter * 2 == n_pairs

    @functools.partial(
        pl.kernel,
        out_type=jax.ShapeDtypeStruct(
            (seq, et, (batch // BLK) * 8 * BLK), jnp.float32),
        mesh=mesh,
        scratch_types=[
            pltpu.VMEM((UPP * BLK,), jnp.int32),
            pltpu.VMEM((UPP * BLK,), jnp.int32),
            pltpu.VMEM((UPP * BLK, embed), jnp.float32),
            pltpu.VMEM((UPP * BLK, embed), jnp.float32),
            pltpu.VMEM((et, UPP * 8 * BLK), jnp.float32),
            pltpu.VMEM((et, UPP * 8 * BLK), jnp.float32),
            pltpu.SemaphoreType.DMA,
            pltpu.SemaphoreType.DMA,
            pltpu.SemaphoreType.DMA,
            pltpu.SemaphoreType.DMA,
            pltpu.SemaphoreType.DMA,
            pltpu.SemaphoreType.DMA,
        ],
        compiler_params=pltpu.CompilerParams(
            use_tc_tiling_on_sc=False, needs_layout_passes=False,
            disable_bounds_checks=True),
    )
    def gather(table_hbm, idxt_hbm, out_hbm, idxb0, idxb1,
               rows0, rows1, tile0, tile1, gs0, gs1, ws0, ws1, is0, is1):
        wid = lax.axis_index("s") * nc + lax.axis_index("c")
        ii = lax.iota(jnp.int32, LANES)

        def pair_coords(p):
            s = p // (ntb // UPP)
            tb0 = UPP * lax.rem(p, ntb // UPP)
            return s, wid * ntb + tb0

        def fetch_idx(p, idxb, isem):
            s, tbg0 = pair_coords(p)
            pltpu.async_copy(
                idxt_hbm.at[s, pl.ds(BLK * tbg0, UPP * BLK)], idxb, isem)

        def wait_idx(idxb, isem):
            pltpu.make_async_copy(
                idxt_hbm.at[0, pl.ds(0, UPP * BLK)], idxb, isem).wait()

        def fire_gathers(idxb, rows, gsem):
            for u in range(UPP):
                pltpu.async_copy(
                    table_hbm.at[idxb.at[pl.ds(u * BLK, BLK)]],
                    rows.at[pl.ds(u * BLK, BLK)], gsem)

        def drain_gathers(rows, gsem):
            pltpu.make_async_copy(
                table_hbm.at[pl.ds(0, UPP * BLK)], rows, gsem).wait()

        pmod = [lax.rem(ii + d, LANES) for d in range(LANES)]
        vrow = [lax.shift_right_logical(pmod[d], 3) for d in range(LANES)]
        vcol = [lax.bitwise_and(pmod[d], 7) * BLK + ii for d in range(LANES)]

        def transpose(rows, tile):
            def blk_body(ug, carry):
                u = ug // (BLK // LANES)
                g = lax.rem(ug, BLK // LANES)
                roff = u * BLK + LANES * g
                coff = u * 8 * BLK + LANES * g
                rvec = roff + ii
                for h in range(embed // LANES):
                    for d in range(LANES):
                        v = plsc.load_gather(
                            rows, [rvec, LANES * h + pmod[d]])
                        plsc.store_scatter(
                            tile, [vrow[d] + 2 * h, vcol[d] + coff], v)
                return carry

            lax.fori_loop(0, UPP * (BLK // LANES), blk_body, 0)

        def fire_write(p, tile, wsem):
            s, tbg0 = pair_coords(p)
            pltpu.async_copy(
                tile, out_hbm.at[s, :, pl.ds(8 * BLK * tbg0, UPP * 8 * BLK)],
                wsem)

        def drain_write(tile, wsem):
            pltpu.make_async_copy(
                out_hbm.at[0, :, pl.ds(0, UPP * 8 * BLK)], tile, wsem).wait()

        fetch_idx(0, idxb0, is0)
        fetch_idx(1, idxb1, is1)
        wait_idx(idxb0, is0)
        fire_gathers(idxb0, rows0, gs0)
        wait_idx(idxb1, is1)
        fire_gathers(idxb1, rows1, gs1)

        def half(t, p, idxb, rows, tile, gsem, wsem, isem):
            drain_gathers(rows, gsem)

            @pl.when(t < n_iter - 1)
            def _():
                fetch_idx(p + 2, idxb, isem)

            @pl.when(t >= 1)
            def _():
                drain_write(tile, wsem)

            transpose(rows, tile)
            fire_write(p, tile, wsem)

            @pl.when(t < n_iter - 1)
            def _():
                wait_idx(idxb, isem)
                fire_gathers(idxb, rows, gsem)

        def body(t, carry):
            p0 = 2 * t
            half(t, p0, idxb0, rows0, tile0, gs0, ws0, is0)
            half(t, p0 + 1, idxb1, rows1, tile1, gs1, ws1, is1)
            return carry

        lax.fori_loop(0, n_iter, body, 0)
        drain_write(tile0, ws0)
        drain_write(tile1, ws1)

    return gather


def kernel(indices, table):
    b, s = indices.shape
    vocab, embed = table.shape
    info = plsc.get_sparse_core_info()
    nc, ns = info.num_cores, info.num_subcores
    idx_t = indices.astype(jnp.int32).T
    out3 = _build_gather(vocab, embed, b, s, nc, ns)(table, idx_t)
    out5 = out3.reshape(s, embed // 8, b // BLK, 8, BLK)
    return jnp.transpose(out5, (2, 4, 0, 1, 3)).reshape(b, s, embed)
